# baseline (speedup 1.0000x reference)
import jax
import jax.numpy as jnp
from jax import lax
from jax.experimental import pallas as pl
from jax.experimental.pallas import tpu as pltpu
from jax.experimental.pallas import tpu_sc as plsc

NC = 2
NS = 16
NW = NC * NS
CHUNK = 128

N = 10000
E = 320000
D = 128
G = 64

ECHUNKS = E // CHUNK
BASE = ECHUNKS // NW
XTRA = ECHUNKS - BASE * NW
ACC_PAD = 10240
APT = ACC_PAD // NS
DEG_PAD = 10240
DPT = DEG_PAD // NS

_MESH = plsc.VectorSubcoreMesh(
    core_axis_name="c", subcore_axis_name="s", num_cores=NC, num_subcores=NS)


def _tile_span(cid, sid):
    wid = sid * NC + cid
    start = wid * BASE + jnp.minimum(wid, XTRA)
    has_extra = wid < XTRA
    return start, has_extra


def _deg_body(es_hbm, deg_out, dstv, onesv, zv, degsh):
    cid = lax.axis_index("c")
    sid = lax.axis_index("s")
    start, has_extra = _tile_span(cid, sid)

    def _fill_ones(r, c):
        onesv[pl.ds(r * 16, 16)] = jnp.ones((16,), jnp.float32)
        return c
    lax.fori_loop(0, CHUNK // 16, _fill_ones, 0)

    def _fill_zero(r, c):
        zv[pl.ds(r * 16, 16)] = jnp.zeros((16,), jnp.float32)
        return c
    lax.fori_loop(0, DPT // 16, _fill_zero, 0)

    pltpu.sync_copy(zv, degsh.at[pl.ds(sid * DPT, DPT)])
    pltpu.sync_copy(es_hbm.at[pl.ds(start, BASE), 1], dstv.at[pl.ds(0, BASE)])

    @pl.when(has_extra)
    def _stage_extra():
        pltpu.sync_copy(es_hbm.at[start + BASE, 1], dstv.at[BASE])

    plsc.subcore_barrier()

    def _scat(j, c):
        pltpu.sync_copy(onesv, degsh.at[dstv.at[j]], add=True)
        return c
    lax.fori_loop(0, BASE, _scat, 0)

    @pl.when(has_extra)
    def _scat_extra():
        pltpu.sync_copy(onesv, degsh.at[dstv.at[BASE]], add=True)

    plsc.subcore_barrier()
    pltpu.sync_copy(degsh.at[pl.ds(sid * DPT, DPT)],
                    deg_out.at[cid, pl.ds(sid * DPT, DPT)])


_deg_call = pl.kernel(
    _deg_body,
    out_type=jax.ShapeDtypeStruct((NC, DEG_PAD), jnp.float32),
    mesh=_MESH,
    scratch_types=[
        pltpu.VMEM((BASE + 1, CHUNK), jnp.int32),
        pltpu.VMEM((CHUNK,), jnp.float32),
        pltpu.VMEM((DPT,), jnp.float32),
        pltpu.VMEM_SHARED((DEG_PAD,), jnp.float32),
    ],
)


def _msg_body(hp_hbm, es_hbm, acc_out, srcv, drow, buf0, buf1,
              accsh, sem0, sem1, semd0, semd1):
    cid = lax.axis_index("c")
    sid = lax.axis_index("s")
    start, has_extra = _tile_span(cid, sid)
    cnt = BASE + has_extra.astype(jnp.int32)

    def _zrow(r, c):
        def _z16(k, c2):
            buf0[r, pl.ds(k * 16, 16)] = jnp.zeros((16,), jnp.float32)
            return c2
        lax.fori_loop(0, D // 16, _z16, 0)
        return c
    lax.fori_loop(0, CHUNK, _zrow, 0)

    def _zcp(t, c):
        pltpu.sync_copy(buf0, accsh.at[pl.ds(sid * APT + t * CHUNK, CHUNK)])
        return c
    lax.fori_loop(0, APT // CHUNK, _zcp, 0)

    pltpu.sync_copy(es_hbm.at[pl.ds(start, BASE), 0], srcv.at[pl.ds(0, BASE)])

    @pl.when(has_extra)
    def _stage_extra():
        pltpu.sync_copy(es_hbm.at[start + BASE, 0], srcv.at[BASE])

    plsc.subcore_barrier()

    pltpu.async_copy(hp_hbm.at[srcv.at[0]], buf0, sem0)
    pltpu.async_copy(es_hbm.at[start, 1], drow.at[0], semd0)
    pltpu.async_copy(hp_hbm.at[srcv.at[1]], buf1, sem1)
    pltpu.async_copy(es_hbm.at[start + 1, 1], drow.at[1], semd1)

    def _step(k, c):
        c0 = k * 2
        c1 = c0 + 1
        pltpu.make_async_copy(hp_hbm.at[srcv.at[c0]], buf0, sem0).wait()
        pltpu.make_async_copy(es_hbm.at[start + c0, 1], drow.at[0],
                              semd0).wait()
        pltpu.sync_copy(buf0, accsh.at[drow.at[0]], add=True)

        @pl.when(c0 + 2 < cnt)
        def _pref0():
            pltpu.async_copy(hp_hbm.at[srcv.at[c0 + 2]], buf0, sem0)
            pltpu.async_copy(es_hbm.at[start + c0 + 2, 1], drow.at[0], semd0)

        pltpu.make_async_copy(hp_hbm.at[srcv.at[c1]], buf1, sem1).wait()
        pltpu.make_async_copy(es_hbm.at[start + c1, 1], drow.at[1],
                              semd1).wait()
        pltpu.sync_copy(buf1, accsh.at[drow.at[1]], add=True)

        @pl.when(c1 + 2 < cnt)
        def _pref1():
            pltpu.async_copy(hp_hbm.at[srcv.at[c1 + 2]], buf1, sem1)
            pltpu.async_copy(es_hbm.at[start + c1 + 2, 1], drow.at[1], semd1)

        return c
    lax.fori_loop(0, BASE // 2, _step, 0)

    @pl.when(has_extra)
    def _tail():
        pltpu.make_async_copy(hp_hbm.at[srcv.at[BASE]], buf0, sem0).wait()
        pltpu.make_async_copy(es_hbm.at[start + BASE, 1], drow.at[0],
                              semd0).wait()
        pltpu.sync_copy(buf0, accsh.at[drow.at[0]], add=True)

    plsc.subcore_barrier()

    def _cpo(t, c):
        r0 = sid * APT + t * CHUNK
        pltpu.sync_copy(accsh.at[pl.ds(r0, CHUNK)],
                        acc_out.at[cid, pl.ds(r0, CHUNK)])
        return c
    lax.fori_loop(0, APT // CHUNK, _cpo, 0)


_msg_call = pl.kernel(
    _msg_body,
    out_type=jax.ShapeDtypeStruct((NC, ACC_PAD, D), jnp.float32),
    mesh=_MESH,
    scratch_types=[
        pltpu.VMEM((BASE + 1, CHUNK), jnp.int32),
        pltpu.VMEM((2, CHUNK), jnp.int32),
        pltpu.VMEM((CHUNK, D), jnp.float32),
        pltpu.VMEM((CHUNK, D), jnp.float32),
        pltpu.VMEM_SHARED((ACC_PAD, D), jnp.float32),
        pltpu.SemaphoreType.DMA,
        pltpu.SemaphoreType.DMA,
        pltpu.SemaphoreType.DMA,
        pltpu.SemaphoreType.DMA,
    ],
)


BLK = 2000
NBLK = N // BLK


def _dinv_col(d_ref, i):
    v = d_ref[:, 0, :] + d_ref[:, 1, :]
    base = i * BLK
    r80 = lax.broadcasted_iota(jnp.int32, (BLK, 80), 0) + base
    c80 = lax.broadcasted_iota(jnp.int32, (BLK, 80), 1)
    pick = jnp.where(c80 == (r80 >> 7), 1.0, 0.0)
    m0 = jnp.dot(pick, v, preferred_element_type=jnp.float32)
    rd = lax.broadcasted_iota(jnp.int32, (BLK, D), 0) + base
    fd = lax.broadcasted_iota(jnp.int32, (BLK, D), 1)
    deg = jnp.sum(jnp.where(fd == (rd & 127), m0, 0.0), axis=1, keepdims=True)
    return 1.0 / jnp.sqrt(deg + 1.0)


_DEG_SPEC = pl.BlockSpec((DEG_PAD // CHUNK, NC, CHUNK), lambda i: (0, 0, 0))


def _tc1a_body(x_ref, w_ref, xw_ref):
    xw_ref[...] = jnp.dot(x_ref[...], w_ref[...],
                          preferred_element_type=jnp.float32)


def _tc1a(x, W1):
    return pl.pallas_call(
        _tc1a_body,
        grid=(NBLK,),
        in_specs=[
            pl.BlockSpec((BLK, D), lambda i: (i, 0)),
            pl.BlockSpec((D, D), lambda i: (0, 0)),
        ],
        out_specs=pl.BlockSpec((BLK, D), lambda i: (i, 0)),
        out_shape=jax.ShapeDtypeStruct((N, D), jnp.float32),
    )(x, W1)


def _tc1b_body(xw_ref, deg_ref, hp_ref):
    dinv = _dinv_col(deg_ref, pl.program_id(0))
    hp_ref[...] = xw_ref[...] * dinv


def _tc1b(xw, degb):
    return pl.pallas_call(
        _tc1b_body,
        grid=(NBLK,),
        in_specs=[
            pl.BlockSpec((BLK, D), lambda i: (i, 0)),
            _DEG_SPEC,
        ],
        out_specs=pl.BlockSpec((BLK, D), lambda i: (i, 0)),
        out_shape=jax.ShapeDtypeStruct((N, D), jnp.float32),
    )(xw, degb)


def _tc2_body(acc_ref, hp_ref, deg_ref, b_ref, w_ref, out_ref):
    dinv = _dinv_col(deg_ref, pl.program_id(0))
    h = (acc_ref[0] + acc_ref[1] + hp_ref[...]) * dinv + b_ref[...][None, :]
    h = jnp.maximum(h, 0.0)
    out_ref[...] = jnp.dot(h, w_ref[...],
                           preferred_element_type=jnp.float32) * dinv


def _tc2(acc, hp, degb, b1, W2):
    return pl.pallas_call(
        _tc2_body,
        grid=(NBLK,),
        in_specs=[
            pl.BlockSpec((NC, BLK, D), lambda i: (0, i, 0)),
            pl.BlockSpec((BLK, D), lambda i: (i, 0)),
            _DEG_SPEC,
            pl.BlockSpec((D,), lambda i: (0,)),
            pl.BlockSpec((D, D), lambda i: (0, 0)),
        ],
        out_specs=pl.BlockSpec((BLK, D), lambda i: (i, 0)),
        out_shape=jax.ShapeDtypeStruct((N, D), jnp.float32),
    )(acc, hp, degb, b1, W2)


def _tc3_body(acc_ref, hp_ref, deg_ref, b_ref, bm_ref, d1_ref, bd1_ref,
              d2_ref, bd2_ref, wct_ref, bc_ref, t_ref, mean_ref, std_ref,
              out_ref, pooled):
    i = pl.program_id(0)

    @pl.when(i == 0)
    def _init():
        pooled[...] = jnp.zeros_like(pooled)

    dinv = _dinv_col(deg_ref, i)
    h = (acc_ref[0] + acc_ref[1] + hp_ref[...]) * dinv + b_ref[...][None, :]
    h = jnp.maximum(h, 0.0)
    bm = bm_ref[0]
    gids = lax.broadcasted_iota(jnp.int32, (G, BLK), 0).astype(jnp.float32)
    onehot = (gids == bm).astype(jnp.float32)
    pooled[...] += jnp.dot(onehot, h, preferred_element_type=jnp.float32)

    @pl.when(i == NBLK - 1)
    def _fin():
        p = jnp.maximum(pooled[...], 0.0)
        h1 = jnp.maximum(
            jnp.dot(p, d1_ref[...], preferred_element_type=jnp.float32)
            + bd1_ref[...][None, :], 0.0)
        h2 = jnp.maximum(
            jnp.dot(h1, d2_ref[...], preferred_element_type=jnp.float32)
            + bd2_ref[...][None, :], 0.0)
        coeff_t = (jnp.dot(wct_ref[...], h2.T,
                           preferred_element_type=jnp.float32)
                   + bc_ref[...])
        a = coeff_t[0]
        b = coeff_t[1]
        cc = coeff_t[2]
        logp = a - b / (t_ref[...] + cc)
        out_ref[...] = (logp - mean_ref[...]) / std_ref[...]


def _tc3(acc, hp, degb, b2, bm_f, D1, bd1, D2, bd2, wc_t, bc2, temperature,
         mean, std):
    return pl.pallas_call(
        _tc3_body,
        grid=(NBLK,),
        in_specs=[
            pl.BlockSpec((NC, BLK, D), lambda i: (0, i, 0)),
            pl.BlockSpec((BLK, D), lambda i: (i, 0)),
            _DEG_SPEC,
            pl.BlockSpec((D,), lambda i: (0,)),
            pl.BlockSpec((1, 1, BLK), lambda i: (i, 0, 0)),
            pl.BlockSpec((D, D), lambda i: (0, 0)),
            pl.BlockSpec((D,), lambda i: (0,)),
            pl.BlockSpec((D, G), lambda i: (0, 0)),
            pl.BlockSpec((G,), lambda i: (0,)),
            pl.BlockSpec((3, G), lambda i: (0, 0)),
            pl.BlockSpec((3, 1), lambda i: (0, 0)),
            pl.BlockSpec((G,), lambda i: (0,)),
            pl.BlockSpec((1,), lambda i: (0,)),
            pl.BlockSpec((1,), lambda i: (0,)),
        ],
        out_specs=pl.BlockSpec((G,), lambda i: (0,)),
        out_shape=jax.ShapeDtypeStruct((G,), jnp.float32),
        scratch_shapes=[pltpu.VMEM((G, D), jnp.float32)],
    )(acc, hp, degb, b2, bm_f, D1, bd1, D2, bd2, wc_t, bc2, temperature,
      mean, std)


def kernel(x, edge_indices, batch_mapping, temperature, mean, std,
           W1, b1, W2, b2, D1, bd1, D2, bd2, Wc, bc):
    f32 = jnp.float32
    es = edge_indices.reshape(2, ECHUNKS, CHUNK).swapaxes(0, 1)
    bm_f = batch_mapping.astype(f32).reshape(NBLK, 1, BLK)
    wc_t = Wc.T
    bc2 = bc.reshape(3, 1)

    degs = _deg_call(es)
    xw1 = _tc1a(x, W1)
    degb = degs.reshape(NC, DEG_PAD // CHUNK, CHUNK).swapaxes(0, 1)
    hp1 = _tc1b(xw1, degb)
    acc1 = _msg_call(hp1, es)
    hp2 = _tc2(acc1, hp1, degb, b1, W2)
    acc2 = _msg_call(hp2, es)
    return _tc3(acc2, hp2, degb, b2, bm_f, D1, bd1, D2, bd2,
                wc_t, bc2, temperature, mean, std)


# speedup vs baseline: 1.0036x; 1.0036x over previous
import jax
import jax.numpy as jnp
from jax import lax
from jax.experimental import pallas as pl
from jax.experimental.pallas import tpu as pltpu
from jax.experimental.pallas import tpu_sc as plsc

NC = 2
NS = 16
NW = NC * NS
CHUNK = 128

N = 10000
E = 320000
D = 128
G = 64

ECHUNKS = E // CHUNK
BASE = ECHUNKS // NW
XTRA = ECHUNKS - BASE * NW
ACC_PAD = 10240
APT = ACC_PAD // NS
DEG_PAD = 10240
DPT = DEG_PAD // NS

_MESH = plsc.VectorSubcoreMesh(
    core_axis_name="c", subcore_axis_name="s", num_cores=NC, num_subcores=NS)


def _tile_span(cid, sid):
    wid = sid * NC + cid
    start = wid * BASE + jnp.minimum(wid, XTRA)
    has_extra = wid < XTRA
    return start, has_extra


def _deg_body(es_hbm, deg_out, dstv, onesv, zv, degsh):
    cid = lax.axis_index("c")
    sid = lax.axis_index("s")
    start, has_extra = _tile_span(cid, sid)

    def _fill_ones(r, c):
        onesv[pl.ds(r * 16, 16)] = jnp.ones((16,), jnp.float32)
        return c
    lax.fori_loop(0, CHUNK // 16, _fill_ones, 0)

    def _fill_zero(r, c):
        zv[pl.ds(r * 16, 16)] = jnp.zeros((16,), jnp.float32)
        return c
    lax.fori_loop(0, DPT // 16, _fill_zero, 0)

    pltpu.sync_copy(zv, degsh.at[pl.ds(sid * DPT, DPT)])
    pltpu.sync_copy(es_hbm.at[pl.ds(start, BASE), 1], dstv.at[pl.ds(0, BASE)])

    @pl.when(has_extra)
    def _stage_extra():
        pltpu.sync_copy(es_hbm.at[start + BASE, 1], dstv.at[BASE])

    plsc.subcore_barrier()

    def _scat(j, c):
        pltpu.sync_copy(onesv, degsh.at[dstv.at[j]], add=True)
        return c
    lax.fori_loop(0, BASE, _scat, 0)

    @pl.when(has_extra)
    def _scat_extra():
        pltpu.sync_copy(onesv, degsh.at[dstv.at[BASE]], add=True)

    plsc.subcore_barrier()
    pltpu.sync_copy(degsh.at[pl.ds(sid * DPT, DPT)],
                    deg_out.at[cid, pl.ds(sid * DPT, DPT)])


_deg_call = pl.kernel(
    _deg_body,
    out_type=jax.ShapeDtypeStruct((NC, DEG_PAD), jnp.float32),
    mesh=_MESH,
    scratch_types=[
        pltpu.VMEM((BASE + 1, CHUNK), jnp.int32),
        pltpu.VMEM((CHUNK,), jnp.float32),
        pltpu.VMEM((DPT,), jnp.float32),
        pltpu.VMEM_SHARED((DEG_PAD,), jnp.float32),
    ],
)


def _msg_body(hp_hbm, es_hbm, acc_out, srcv, drow, buf0, buf1,
              accsh, sem0, sem1, semd0, semd1):
    cid = lax.axis_index("c")
    sid = lax.axis_index("s")
    start, has_extra = _tile_span(cid, sid)
    cnt = BASE + has_extra.astype(jnp.int32)

    def _zrow(r, c):
        def _z16(k, c2):
            buf0[r, pl.ds(k * 16, 16)] = jnp.zeros((16,), jnp.float32)
            return c2
        lax.fori_loop(0, D // 16, _z16, 0)
        return c
    lax.fori_loop(0, CHUNK, _zrow, 0)

    def _zcp(t, c):
        pltpu.sync_copy(buf0, accsh.at[pl.ds(sid * APT + t * CHUNK, CHUNK)])
        return c
    lax.fori_loop(0, APT // CHUNK, _zcp, 0)

    pltpu.sync_copy(es_hbm.at[pl.ds(start, BASE), 0], srcv.at[pl.ds(0, BASE)])

    @pl.when(has_extra)
    def _stage_extra():
        pltpu.sync_copy(es_hbm.at[start + BASE, 0], srcv.at[BASE])

    plsc.subcore_barrier()

    pltpu.async_copy(hp_hbm.at[srcv.at[0]], buf0, sem0)
    pltpu.async_copy(es_hbm.at[start, 1], drow.at[0], semd0)
    pltpu.async_copy(hp_hbm.at[srcv.at[1]], buf1, sem1)
    pltpu.async_copy(es_hbm.at[start + 1, 1], drow.at[1], semd1)

    def _step(k, c):
        c0 = k * 2
        c1 = c0 + 1
        pltpu.make_async_copy(hp_hbm.at[srcv.at[c0]], buf0, sem0).wait()
        pltpu.make_async_copy(es_hbm.at[start + c0, 1], drow.at[0],
                              semd0).wait()
        pltpu.sync_copy(buf0, accsh.at[drow.at[0]], add=True)

        @pl.when(c0 + 2 < cnt)
        def _pref0():
            pltpu.async_copy(hp_hbm.at[srcv.at[c0 + 2]], buf0, sem0)
            pltpu.async_copy(es_hbm.at[start + c0 + 2, 1], drow.at[0], semd0)

        pltpu.make_async_copy(hp_hbm.at[srcv.at[c1]], buf1, sem1).wait()
        pltpu.make_async_copy(es_hbm.at[start + c1, 1], drow.at[1],
                              semd1).wait()
        pltpu.sync_copy(buf1, accsh.at[drow.at[1]], add=True)

        @pl.when(c1 + 2 < cnt)
        def _pref1():
            pltpu.async_copy(hp_hbm.at[srcv.at[c1 + 2]], buf1, sem1)
            pltpu.async_copy(es_hbm.at[start + c1 + 2, 1], drow.at[1], semd1)

        return c
    lax.fori_loop(0, BASE // 2, _step, 0)

    @pl.when(has_extra)
    def _tail():
        pltpu.make_async_copy(hp_hbm.at[srcv.at[BASE]], buf0, sem0).wait()
        pltpu.make_async_copy(es_hbm.at[start + BASE, 1], drow.at[0],
                              semd0).wait()
        pltpu.sync_copy(buf0, accsh.at[drow.at[0]], add=True)

    plsc.subcore_barrier()

    def _cpo(t, c):
        r0 = sid * APT + t * CHUNK
        pltpu.sync_copy(accsh.at[pl.ds(r0, CHUNK)],
                        acc_out.at[cid, pl.ds(r0, CHUNK)])
        return c
    lax.fori_loop(0, APT // CHUNK, _cpo, 0)


_msg_call = pl.kernel(
    _msg_body,
    out_type=jax.ShapeDtypeStruct((NC, ACC_PAD, D), jnp.float32),
    mesh=_MESH,
    scratch_types=[
        pltpu.VMEM((BASE + 1, CHUNK), jnp.int32),
        pltpu.VMEM((2, CHUNK), jnp.int32),
        pltpu.VMEM((CHUNK, D), jnp.float32),
        pltpu.VMEM((CHUNK, D), jnp.float32),
        pltpu.VMEM_SHARED((ACC_PAD, D), jnp.float32),
        pltpu.SemaphoreType.DMA,
        pltpu.SemaphoreType.DMA,
        pltpu.SemaphoreType.DMA,
        pltpu.SemaphoreType.DMA,
    ],
)


BLK = 2000
NBLK = N // BLK


def _dinv_col(d_ref, i):
    v = d_ref[:, 0, :] + d_ref[:, 1, :]
    base = i * BLK
    r80 = lax.broadcasted_iota(jnp.int32, (BLK, 80), 0) + base
    c80 = lax.broadcasted_iota(jnp.int32, (BLK, 80), 1)
    pick = jnp.where(c80 == (r80 >> 7), 1.0, 0.0)
    m0 = jnp.dot(pick, v, preferred_element_type=jnp.float32)
    rd = lax.broadcasted_iota(jnp.int32, (BLK, D), 0) + base
    fd = lax.broadcasted_iota(jnp.int32, (BLK, D), 1)
    deg = jnp.sum(jnp.where(fd == (rd & 127), m0, 0.0), axis=1, keepdims=True)
    return 1.0 / jnp.sqrt(deg + 1.0)


_DEG_SPEC = pl.BlockSpec((DEG_PAD // CHUNK, NC, CHUNK), lambda i: (0, 0, 0))


def _tc1_body(x_ref, w_ref, deg_ref, hp_ref):
    dinv = _dinv_col(deg_ref, pl.program_id(0))
    hp_ref[...] = jnp.dot(x_ref[...], w_ref[...],
                          preferred_element_type=jnp.float32) * dinv


def _tc1(x, W1, degb):
    return pl.pallas_call(
        _tc1_body,
        grid=(NBLK,),
        in_specs=[
            pl.BlockSpec((BLK, D), lambda i: (i, 0)),
            pl.BlockSpec((D, D), lambda i: (0, 0)),
            _DEG_SPEC,
        ],
        out_specs=pl.BlockSpec((BLK, D), lambda i: (i, 0)),
        out_shape=jax.ShapeDtypeStruct((N, D), jnp.float32),
    )(x, W1, degb)


def _tc2_body(acc_ref, hp_ref, deg_ref, b_ref, w_ref, out_ref):
    dinv = _dinv_col(deg_ref, pl.program_id(0))
    h = (acc_ref[0] + acc_ref[1] + hp_ref[...]) * dinv + b_ref[...][None, :]
    h = jnp.maximum(h, 0.0)
    out_ref[...] = jnp.dot(h, w_ref[...],
                           preferred_element_type=jnp.float32) * dinv


def _tc2(acc, hp, degb, b1, W2):
    return pl.pallas_call(
        _tc2_body,
        grid=(NBLK,),
        in_specs=[
            pl.BlockSpec((NC, BLK, D), lambda i: (0, i, 0)),
            pl.BlockSpec((BLK, D), lambda i: (i, 0)),
            _DEG_SPEC,
            pl.BlockSpec((D,), lambda i: (0,)),
            pl.BlockSpec((D, D), lambda i: (0, 0)),
        ],
        out_specs=pl.BlockSpec((BLK, D), lambda i: (i, 0)),
        out_shape=jax.ShapeDtypeStruct((N, D), jnp.float32),
    )(acc, hp, degb, b1, W2)


def _tc3_body(acc_ref, hp_ref, deg_ref, b_ref, bm_ref, d1_ref, bd1_ref,
              d2_ref, bd2_ref, wct_ref, bc_ref, t_ref, mean_ref, std_ref,
              out_ref, pooled):
    i = pl.program_id(0)

    @pl.when(i == 0)
    def _init():
        pooled[...] = jnp.zeros_like(pooled)

    dinv = _dinv_col(deg_ref, i)
    h = (acc_ref[0] + acc_ref[1] + hp_ref[...]) * dinv + b_ref[...][None, :]
    h = jnp.maximum(h, 0.0)
    bm = bm_ref[0]
    gids = lax.broadcasted_iota(jnp.int32, (G, BLK), 0).astype(jnp.float32)
    onehot = (gids == bm).astype(jnp.float32)
    pooled[...] += jnp.dot(onehot, h, preferred_element_type=jnp.float32)

    @pl.when(i == NBLK - 1)
    def _fin():
        p = jnp.maximum(pooled[...], 0.0)
        h1 = jnp.maximum(
            jnp.dot(p, d1_ref[...], preferred_element_type=jnp.float32)
            + bd1_ref[...][None, :], 0.0)
        h2 = jnp.maximum(
            jnp.dot(h1, d2_ref[...], preferred_element_type=jnp.float32)
            + bd2_ref[...][None, :], 0.0)
        coeff_t = (jnp.dot(wct_ref[...], h2.T,
                           preferred_element_type=jnp.float32)
                   + bc_ref[...])
        a = coeff_t[0]
        b = coeff_t[1]
        cc = coeff_t[2]
        logp = a - b / (t_ref[...] + cc)
        out_ref[...] = (logp - mean_ref[...]) / std_ref[...]


def _tc3(acc, hp, degb, b2, bm_f, D1, bd1, D2, bd2, wc_t, bc2, temperature,
         mean, std):
    return pl.pallas_call(
        _tc3_body,
        grid=(NBLK,),
        in_specs=[
            pl.BlockSpec((NC, BLK, D), lambda i: (0, i, 0)),
            pl.BlockSpec((BLK, D), lambda i: (i, 0)),
            _DEG_SPEC,
            pl.BlockSpec((D,), lambda i: (0,)),
            pl.BlockSpec((1, 1, BLK), lambda i: (i, 0, 0)),
            pl.BlockSpec((D, D), lambda i: (0, 0)),
            pl.BlockSpec((D,), lambda i: (0,)),
            pl.BlockSpec((D, G), lambda i: (0, 0)),
            pl.BlockSpec((G,), lambda i: (0,)),
            pl.BlockSpec((3, G), lambda i: (0, 0)),
            pl.BlockSpec((3, 1), lambda i: (0, 0)),
            pl.BlockSpec((G,), lambda i: (0,)),
            pl.BlockSpec((1,), lambda i: (0,)),
            pl.BlockSpec((1,), lambda i: (0,)),
        ],
        out_specs=pl.BlockSpec((G,), lambda i: (0,)),
        out_shape=jax.ShapeDtypeStruct((G,), jnp.float32),
        scratch_shapes=[pltpu.VMEM((G, D), jnp.float32)],
    )(acc, hp, degb, b2, bm_f, D1, bd1, D2, bd2, wc_t, bc2, temperature,
      mean, std)


def kernel(x, edge_indices, batch_mapping, temperature, mean, std,
           W1, b1, W2, b2, D1, bd1, D2, bd2, Wc, bc):
    f32 = jnp.float32
    es = edge_indices.reshape(2, ECHUNKS, CHUNK).swapaxes(0, 1)
    bm_f = batch_mapping.astype(f32).reshape(NBLK, 1, BLK)
    wc_t = Wc.T
    bc2 = bc.reshape(3, 1)

    degs = _deg_call(es)
    degb = degs.reshape(NC, DEG_PAD // CHUNK, CHUNK).swapaxes(0, 1)
    hp1 = _tc1(x, W1, degb)
    acc1 = _msg_call(hp1, es)
    hp2 = _tc2(acc1, hp1, degb, b1, W2)
    acc2 = _msg_call(hp2, es)
    return _tc3(acc2, hp2, degb, b2, bm_f, D1, bd1, D2, bd2,
                wc_t, bc2, temperature, mean, std)
